# baseline (speedup 1.0000x reference)
import functools

import jax
import jax.numpy as jnp
from jax import lax
from jax.experimental import pallas as pl
from jax.experimental.pallas import tpu as pltpu
from jax.experimental.pallas import tpu_sc as plsc

N = 50000
F_X = 256
F_U = 128
F_OUT = 128
B = 512

NC = 2
NS = 16
NW = NC * NS

BLK = 128
NFULL = (N // BLK)
TAIL = N - NFULL * BLK
BASE_CNT = NFULL // NW
EXTRA = NFULL - BASE_CNT * NW
MAXB = BASE_CNT + 1


HF = F_X // NC
LANES = 16
HCHUNK = HF // LANES


def _sc_segsum(x, batch):
    mesh = plsc.VectorSubcoreMesh(core_axis_name="c", subcore_axis_name="s")

    s_cnt = 24
    s_extra = 3
    maxc = s_cnt + 2

    @functools.partial(
        pl.kernel,
        out_type=jax.ShapeDtypeStruct((NC, NS, B, HF), jnp.float32),
        mesh=mesh,
        scratch_types=[
            pltpu.VMEM((maxc * BLK,), jnp.int32),
            pltpu.VMEM((BLK, HF), jnp.float32),
            pltpu.VMEM((BLK, HF), jnp.float32),
            pltpu.VMEM((B, HF), jnp.float32),
            pltpu.SemaphoreType.DMA,
            pltpu.SemaphoreType.DMA,
            pltpu.SemaphoreType.DMA,
        ],
    )
    def sc_kernel(x_hbm, batch_hbm, part_hbm, idx_all, xb0, xb1, acc,
                  sem_i, sem0, sem1):
        c = lax.axis_index("c")
        s = lax.axis_index("s")
        co = pl.multiple_of(c * HF, HF)

        base = s * s_cnt + 2 * jnp.minimum(s, s_extra)
        cnt = s_cnt + 2 * (s < s_extra).astype(jnp.int32)
        base_off = pl.multiple_of(base * BLK, BLK)

        idx_cp = pltpu.async_copy(
            batch_hbm.at[pl.ds(base_off, maxc * BLK)], idx_all, sem_i)
        pltpu.async_copy(
            x_hbm.at[pl.ds(base_off, BLK), pl.ds(co, HF)], xb0, sem0)

        def zero_body(r, carry):
            for k in range(HCHUNK):
                acc[r, pl.ds(k * LANES, LANES)] = jnp.zeros((LANES,),
                                                            jnp.float32)
            return carry

        lax.fori_loop(0, B, zero_body, 0)
        idx_cp.wait()

        def accumulate(i, buf, csem, nxt, nsem, carry):
            pltpu.make_async_copy(
                x_hbm.at[pl.ds(0, BLK), pl.ds(0, HF)], buf, csem).wait()

            @pl.when(i + 1 < cnt)
            def _():
                off = pl.multiple_of((base + i + 1) * BLK, BLK)
                pltpu.async_copy(
                    x_hbm.at[pl.ds(off, BLK), pl.ds(co, HF)], nxt, nsem)

            def grp_body(g, carry2):
                r0 = pl.multiple_of(g * LANES, LANES)
                io = pl.multiple_of(i * BLK + r0, LANES)
                segv = idx_all[pl.ds(io, LANES)]
                s0 = segv[0]
                s15 = segv[LANES - 1]

                @pl.when(s0 == s15)
                def _():
                    for k in range(HCHUNK):
                        dk = pl.ds(k * LANES, LANES)
                        v = [buf[r0 + j, dk] for j in range(LANES)]
                        while len(v) > 1:
                            v = [v[m] + v[m + 1]
                                 for m in range(0, len(v) - 1, 2)] + (
                                     [v[-1]] if len(v) % 2 else [])
                        plsc.addupdate(acc.at[s0, dk], v[0])

                @pl.when(s0 != s15)
                def _():
                    for j in range(LANES):
                        seg = segv[j]
                        for k in range(HCHUNK):
                            plsc.addupdate(
                                acc.at[seg, pl.ds(k * LANES, LANES)],
                                buf[r0 + j, pl.ds(k * LANES, LANES)])
                return carry2

            return lax.fori_loop(0, BLK // LANES, grp_body, carry)

        def pair_body(p, carry):
            i0 = p * 2
            carry = accumulate(i0, xb0, sem0, xb1, sem1, carry)
            carry = accumulate(i0 + 1, xb1, sem1, xb0, sem0, carry)
            return carry

        lax.fori_loop(0, cnt // 2, pair_body, 0)

        pltpu.sync_copy(acc, part_hbm.at[c, s])

    return sc_kernel(x, batch)


def _tc_body(p_ref, xt_ref, bt_ref, u_ref, w_ref, b_ref, o_ref, agg_ref):
    i = pl.program_id(0)

    @pl.when(i == 0)
    def _():
        agg_ref[...] = jnp.zeros_like(agg_ref)

    agg_ref[:, :HF] += p_ref[0, 0]
    agg_ref[:, HF:] += p_ref[1, 0]

    @pl.when(i == pl.num_programs(0) - 1)
    def _():
        bt = bt_ref[0, :]
        onehot = (lax.broadcasted_iota(jnp.int32, (B, TAIL), 0)
                  == bt[None, :]).astype(jnp.float32)
        agg = agg_ref[...] + jnp.dot(onehot, xt_ref[...],
                                     preferred_element_type=jnp.float32)
        out = jnp.dot(agg, w_ref[:F_X, :], preferred_element_type=jnp.float32)
        out = out + jnp.dot(u_ref[...], w_ref[F_X:, :],
                            preferred_element_type=jnp.float32)
        o_ref[...] = out + b_ref[0, :][None, :]


def kernel(x, u, batch, W, b):
    batch = batch.astype(jnp.int32)
    batch_pad = jnp.concatenate([batch, jnp.zeros((176,), jnp.int32)])

    parts = _sc_segsum(x, batch_pad)

    x_tail = x[NFULL * BLK:]
    bt = jnp.broadcast_to(batch[NFULL * BLK:][None, :], (8, TAIL))

    full = lambda *shape: pl.BlockSpec(shape, lambda i: (0,) * len(shape))
    return pl.pallas_call(
        _tc_body,
        grid=(NS,),
        in_specs=[
            pl.BlockSpec((NC, 1, B, HF), lambda i: (0, i, 0, 0)),
            full(TAIL, F_X),
            full(8, TAIL),
            full(B, F_U),
            full(F_X + F_U, F_OUT),
            full(1, F_OUT),
        ],
        out_specs=full(B, F_OUT),
        scratch_shapes=[pltpu.VMEM((B, F_X), jnp.float32)],
        out_shape=jax.ShapeDtypeStruct((B, F_OUT), jnp.float32),
    )(parts, x_tail, bt, u, W, b.reshape(1, F_OUT))
